# baseline (speedup 1.0000x reference)
import functools

import jax
import jax.numpy as jnp
from jax import lax
from jax.experimental import pallas as pl
from jax.experimental.pallas import tpu as pltpu
from jax.experimental.pallas import tpu_sc as plsc

NC = 2
NS = 16
NW = NC * NS
CHUNK = 128


def _rowmajor(table):
    v = table.shape[0]
    packed = jnp.reshape(table, (v // 4, 128))
    return lax.optimization_barrier(packed).reshape(v, 32)


def kernel(table0, table1, table2, table3, indexes):
    B = indexes.shape[0]
    D3 = table3.shape[1]
    OUT_D = 128
    bpw = B // NW
    nch = bpw // CHUNK

    idxr = indexes.astype(jnp.int32).reshape(NW, nch, CHUNK)
    t0 = _rowmajor(table0)
    t1 = _rowmajor(table1)
    t2 = _rowmajor(table2)
    t3 = _rowmajor(jnp.pad(table3, ((0, 0), (0, 32 - D3))))

    mesh = plsc.VectorSubcoreMesh(core_axis_name="c", subcore_axis_name="s")

    @functools.partial(
        pl.kernel,
        mesh=mesh,
        out_type=jax.ShapeDtypeStruct((B, OUT_D), jnp.float32),
        compiler_params=pltpu.CompilerParams(
            use_tc_tiling_on_sc=False, needs_layout_passes=False),
        scratch_types=[
            pltpu.VMEM((nch, CHUNK), jnp.int32),
            pltpu.VMEM((bpw, 32), jnp.float32),
            pltpu.VMEM((bpw, 32), jnp.float32),
            pltpu.VMEM((bpw, 32), jnp.float32),
            pltpu.VMEM((bpw, 32), jnp.float32),
            pltpu.SemaphoreType.DMA,
        ],
    )
    def sc_kernel(t0h, t1h, t2h, t3h, idx_hbm, out_hbm,
                  idx_v, b0, b1, b2, b3, sem):
        wid = lax.axis_index("s") * NC + lax.axis_index("c")
        base = wid * bpw
        pltpu.sync_copy(idx_hbm.at[wid], idx_v)
        cps = []
        for j in range(nch):
            rows = pl.ds(j * CHUNK, CHUNK)
            ij = idx_v.at[j]
            cps.append(pltpu.async_copy(t0h.at[ij], b0.at[rows], sem))
            cps.append(pltpu.async_copy(t1h.at[ij], b1.at[rows], sem))
            cps.append(pltpu.async_copy(t2h.at[ij], b2.at[rows], sem))
            cps.append(pltpu.async_copy(t3h.at[ij], b3.at[rows], sem))
        for c in cps:
            c.wait()
        orows = pl.ds(base, bpw)
        pltpu.sync_copy(b0, out_hbm.at[orows, pl.ds(0, 32)])
        pltpu.sync_copy(b1, out_hbm.at[orows, pl.ds(32, 32)])
        pltpu.sync_copy(b2, out_hbm.at[orows, pl.ds(64, 32)])
        pltpu.sync_copy(b3, out_hbm.at[orows, pl.ds(96, 32)])

    return sc_kernel(t0, t1, t2, t3, idxr)


# speedup vs baseline: 1.0059x; 1.0059x over previous
import functools

import jax
import jax.numpy as jnp
from jax import lax
from jax.experimental import pallas as pl
from jax.experimental.pallas import tpu as pltpu
from jax.experimental.pallas import tpu_sc as plsc

NC = 2
NS = 16
NW = NC * NS
CHUNK = 128


def _rowmajor(table):
    tt = lax.optimization_barrier(jnp.swapaxes(table, 0, 1))
    return lax.optimization_barrier(jnp.swapaxes(tt, 0, 1))


def kernel(table0, table1, table2, table3, indexes):
    B = indexes.shape[0]
    D3 = table3.shape[1]
    OUT_D = 128
    bpw = B // NW
    nch = bpw // CHUNK

    idxr = indexes.astype(jnp.int32).reshape(NW, nch, CHUNK)
    t0 = _rowmajor(table0)
    t1 = _rowmajor(table1)
    t2 = _rowmajor(table2)
    t3 = _rowmajor(jnp.pad(table3, ((0, 0), (0, 32 - D3))))

    mesh = plsc.VectorSubcoreMesh(core_axis_name="c", subcore_axis_name="s")

    @functools.partial(
        pl.kernel,
        mesh=mesh,
        out_type=jax.ShapeDtypeStruct((B, OUT_D), jnp.float32),
        compiler_params=pltpu.CompilerParams(
            use_tc_tiling_on_sc=False, needs_layout_passes=False),
        scratch_types=[
            pltpu.VMEM((nch, CHUNK), jnp.int32),
            pltpu.VMEM((bpw, 32), jnp.float32),
            pltpu.VMEM((bpw, 32), jnp.float32),
            pltpu.VMEM((bpw, 32), jnp.float32),
            pltpu.VMEM((bpw, 32), jnp.float32),
            pltpu.SemaphoreType.DMA,
        ],
    )
    def sc_kernel(t0h, t1h, t2h, t3h, idx_hbm, out_hbm,
                  idx_v, b0, b1, b2, b3, sem):
        wid = lax.axis_index("s") * NC + lax.axis_index("c")
        base = wid * bpw
        pltpu.sync_copy(idx_hbm.at[wid], idx_v)
        cps = []
        for j in range(nch):
            rows = pl.ds(j * CHUNK, CHUNK)
            ij = idx_v.at[j]
            cps.append(pltpu.async_copy(t0h.at[ij], b0.at[rows], sem))
            cps.append(pltpu.async_copy(t1h.at[ij], b1.at[rows], sem))
            cps.append(pltpu.async_copy(t2h.at[ij], b2.at[rows], sem))
            cps.append(pltpu.async_copy(t3h.at[ij], b3.at[rows], sem))
        for c in cps:
            c.wait()
        orows = pl.ds(base, bpw)
        pltpu.sync_copy(b0, out_hbm.at[orows, pl.ds(0, 32)])
        pltpu.sync_copy(b1, out_hbm.at[orows, pl.ds(32, 32)])
        pltpu.sync_copy(b2, out_hbm.at[orows, pl.ds(64, 32)])
        pltpu.sync_copy(b3, out_hbm.at[orows, pl.ds(96, 32)])

    return sc_kernel(t0, t1, t2, t3, idxr)
